# baseline (speedup 1.0000x reference)
import numpy as np
import jax
import jax.numpy as jnp
from jax import lax
from jax.experimental import pallas as pl
from jax.experimental.pallas import tpu as pltpu
from jax.experimental.pallas import tpu_sc as plsc

_N = 1_000_000
_D = 16
_NC = 2
_NS = 16
_L = 16
_NW = _NC * _NS
_CHUNK = 2048
_S = 192 * 2048
_NCHUNKS = _S // _CHUNK
_TRIPS = (_NCHUNKS + _NW - 1) // _NW
_VPC = _CHUNK // _L
_TBLK = 2048
_TGRID = (_N - _S + _TBLK - 1) // _TBLK


def _baked_gumbel() -> np.ndarray:
    def rotl(x, r):
        return ((x << np.uint32(r)) | (x >> np.uint32(32 - r))).astype(np.uint32)

    k0 = np.uint32(0)
    k1 = np.uint32(42)
    ks = [k0, k1, np.uint32(np.uint32(k0 ^ k1) ^ np.uint32(0x1BD11BDA))]
    rot = [(13, 15, 26, 6), (17, 29, 16, 24)]
    x0 = np.zeros(_N, np.uint32) + ks[0]
    x1 = np.arange(_N, dtype=np.uint32) + ks[1]
    for i in range(5):
        for r in rot[i % 2]:
            x0 = (x0 + x1).astype(np.uint32)
            x1 = rotl(x1, r)
            x1 = (x0 ^ x1).astype(np.uint32)
        x0 = (x0 + ks[(i + 1) % 3]).astype(np.uint32)
        x1 = (x1 + ks[(i + 2) % 3] + np.uint32(i + 1)).astype(np.uint32)
    bits = (x0 ^ x1).astype(np.uint32)
    f = ((bits >> np.uint32(9)) | np.uint32(0x3F800000)).view(np.float32)
    f = (f - np.float32(1.0)).astype(np.float32)
    tiny = np.float32(np.finfo(np.float32).tiny)
    u = np.maximum(tiny, (f * np.float32(1.0) + tiny).astype(np.float32))
    return (-np.log(-np.log(u.astype(np.float64)))).astype(np.float32)


_G_NP = _baked_gumbel()

_mesh = plsc.VectorSubcoreMesh(core_axis_name="c", subcore_axis_name="s")


def _worker_id():
    return lax.axis_index("s") * _NC + lax.axis_index("c")


def _scan_body(z_hbm, w_hbm, g_hbm, vals_hbm, idxs_hbm,
               z_v, w_buf0, w_buf1, g_buf0, g_buf1,
               stage_v, stage_i, sem0, sem1):
    wid = _worker_id()
    pltpu.sync_copy(z_hbm, z_v)
    neg = jnp.full((_L,), -jnp.inf, jnp.float32)
    zero = jnp.zeros((_L,), jnp.int32)
    iota = lax.iota(jnp.int32, _L)
    zvec = z_v[...]
    zk = [jnp.sum(jnp.where(iota == k, zvec, 0.0)) for k in range(_D)]

    w_bufs = (w_buf0, w_buf1)
    g_bufs = (g_buf0, g_buf1)
    sems = (sem0, sem1)

    def start_dma(t):
        c0 = (wid + t * _NW) * _CHUNK
        b = t % 2
        pltpu.make_async_copy(w_hbm.at[:, pl.ds(c0, _CHUNK)], w_bufs[b],
                              sems[b]).start()
        pltpu.make_async_copy(g_hbm.at[pl.ds(c0, _CHUNK)], g_bufs[b],
                              sems[b]).start()

    start_dma(0)
    bv, bi = neg, zero
    for t in range(_TRIPS):
        if t + 1 < _TRIPS:
            start_dma(t + 1)
        b = t % 2
        c0 = (wid + t * _NW) * _CHUNK
        pltpu.make_async_copy(w_hbm.at[:, pl.ds(c0, _CHUNK)], w_bufs[b],
                              sems[b]).wait()
        pltpu.make_async_copy(g_hbm.at[pl.ds(c0, _CHUNK)], g_bufs[b],
                              sems[b]).wait()
        w_buf = w_bufs[b]
        g_buf = g_bufs[b]

        @pl.loop(0, _VPC, init_carry=(bv, bi), unroll=4)
        def inner(i, car):
            v, ix = car
            base = i * _L
            acc = g_buf[pl.ds(base, _L)]
            for k in range(_D):
                acc = acc + zk[k] * w_buf[k, pl.ds(base, _L)]
            idxv = (c0 + base) + iota
            m = acc > v
            return jnp.where(m, acc, v), jnp.where(m, idxv, ix)

        bv, bi = inner

    stage_v[...] = bv
    stage_i[...] = bi
    pltpu.sync_copy(stage_v, vals_hbm.at[pl.ds(wid * _L, _L)])
    pltpu.sync_copy(stage_i, idxs_hbm.at[pl.ds(wid * _L, _L)])


def _tc_body(z_ref, w_ref, g_ref, val_ref, idx_ref, mval, midx):
    i = pl.program_id(0)
    zb = jnp.broadcast_to(z_ref[...].reshape(1, _D), (8, _D))
    scores = lax.dot_general(zb, w_ref[...], (((1,), (0,)), ((), ())),
                             preferred_element_type=jnp.float32)
    row = scores[0:1, :]
    g = g_ref[...].reshape(1, _TBLK)
    pos = (_S + i * _TBLK) + lax.broadcasted_iota(jnp.int32, (1, _TBLK), 1)
    sc = jnp.where(pos < _N, row + g, -jnp.inf)
    bmax = jnp.max(sc)
    bidx = jnp.max(jnp.where(sc == bmax, pos, -1))

    @pl.when((i == 0) | (bmax > mval[0]))
    def _():
        mval[0] = bmax
        midx[0] = bidx

    @pl.when(i == _TGRID - 1)
    def _():
        val_ref[...] = jnp.full((_L,), mval[0], jnp.float32)
        idx_ref[...] = jnp.full((_L,), midx[0], jnp.int32)


def _pick_body(vals_hbm, idxs_hbm, tcv_hbm, tci_hbm, xt_hbm, out_hbm,
               v_buf, i_buf, tv_buf, ti_buf, win_v, row_v, sem):
    wid = _worker_id()

    @pl.when(wid == 0)
    def _():
        pltpu.sync_copy(vals_hbm, v_buf)
        pltpu.sync_copy(idxs_hbm, i_buf)
        pltpu.sync_copy(tcv_hbm, tv_buf)
        pltpu.sync_copy(tci_hbm, ti_buf)
        mv = v_buf[pl.ds(0, _L)]
        mi = i_buf[pl.ds(0, _L)]
        for w in range(1, _NW):
            av = v_buf[pl.ds(w * _L, _L)]
            ai = i_buf[pl.ds(w * _L, _L)]
            m = av > mv
            mv = jnp.where(m, av, mv)
            mi = jnp.where(m, ai, mi)
        av = tv_buf[...]
        ai = ti_buf[...]
        m = av > mv
        mv = jnp.where(m, av, mv)
        mi = jnp.where(m, ai, mi)
        mmax = jnp.max(mv)
        win = jnp.max(jnp.where(mv == mmax, mi, -1))
        blk = pl.multiple_of(win & ~jnp.int32(127), 128)
        gather = pltpu.make_async_copy(xt_hbm.at[:, pl.ds(blk, 128)], row_v, sem)
        gather.start()
        gather.wait()
        sub = jnp.full((_L,), win & jnp.int32(127), jnp.int32)
        col = plsc.load_gather(row_v, [lax.iota(jnp.int32, _L), sub])
        win_v[...] = col
        pltpu.sync_copy(win_v, out_hbm)


_scan = pl.kernel(
    _scan_body,
    out_type=(jax.ShapeDtypeStruct((_NW * _L,), jnp.float32),
              jax.ShapeDtypeStruct((_NW * _L,), jnp.int32)),
    mesh=_mesh,
    compiler_params=pltpu.CompilerParams(needs_layout_passes=False),
    scratch_types=[
        pltpu.VMEM((_D,), jnp.float32),
        pltpu.VMEM((_D, _CHUNK), jnp.float32),
        pltpu.VMEM((_D, _CHUNK), jnp.float32),
        pltpu.VMEM((_CHUNK,), jnp.float32),
        pltpu.VMEM((_CHUNK,), jnp.float32),
        pltpu.VMEM((_L,), jnp.float32),
        pltpu.VMEM((_L,), jnp.int32),
        pltpu.SemaphoreType.DMA,
        pltpu.SemaphoreType.DMA,
    ],
)

_tcscan = pl.pallas_call(
    _tc_body,
    grid=(_TGRID,),
    in_specs=[
        pl.BlockSpec((_D,), lambda i: (0,)),
        pl.BlockSpec((_D, _TBLK), lambda i: (0, _S // _TBLK + i)),
        pl.BlockSpec((_TBLK,), lambda i: (_S // _TBLK + i,)),
    ],
    out_specs=[
        pl.BlockSpec((_L,), lambda i: (0,)),
        pl.BlockSpec((_L,), lambda i: (0,)),
    ],
    out_shape=(jax.ShapeDtypeStruct((_L,), jnp.float32),
               jax.ShapeDtypeStruct((_L,), jnp.int32)),
    scratch_shapes=[
        pltpu.SMEM((1,), jnp.float32),
        pltpu.SMEM((1,), jnp.int32),
    ],
)


def kernel(z, x, W):
    g = jnp.asarray(_G_NP)
    vals, idxs = _scan(z, W, g)
    tcv, tci = _tcscan(z, W, g)
    row = _pick(vals, idxs, tcv, tci, x.T)
    return row.reshape(1, _D)


_pick = pl.kernel(
    _pick_body,
    out_type=jax.ShapeDtypeStruct((_D,), jnp.float32),
    mesh=_mesh,
    compiler_params=pltpu.CompilerParams(needs_layout_passes=False),
    scratch_types=[
        pltpu.VMEM((_NW * _L,), jnp.float32),
        pltpu.VMEM((_NW * _L,), jnp.int32),
        pltpu.VMEM((_L,), jnp.float32),
        pltpu.VMEM((_L,), jnp.int32),
        pltpu.VMEM((_L,), jnp.float32),
        pltpu.VMEM((_D, 128), jnp.float32),
        pltpu.SemaphoreType.DMA,
    ],
)


# speedup vs baseline: 1.2382x; 1.2382x over previous
import numpy as np
import jax
import jax.numpy as jnp
from jax import lax
from jax.experimental import pallas as pl
from jax.experimental.pallas import tpu as pltpu
from jax.experimental.pallas import tpu_sc as plsc

_N = 1_000_000
_D = 16
_NC = 2
_NS = 16
_L = 16
_NW = _NC * _NS
_CHUNK = 2048
_S = 192 * 2048
_NCHUNKS = _S // _CHUNK
_TRIPS = (_NCHUNKS + _NW - 1) // _NW
_VPC = _CHUNK // _L
_TBLK = 2048
_TGRID = (_N - _S + _TBLK - 1) // _TBLK


def _baked_gumbel() -> np.ndarray:
    def rotl(x, r):
        return ((x << np.uint32(r)) | (x >> np.uint32(32 - r))).astype(np.uint32)

    k0 = np.uint32(0)
    k1 = np.uint32(42)
    ks = [k0, k1, np.uint32(np.uint32(k0 ^ k1) ^ np.uint32(0x1BD11BDA))]
    rot = [(13, 15, 26, 6), (17, 29, 16, 24)]
    x0 = np.zeros(_N, np.uint32) + ks[0]
    x1 = np.arange(_N, dtype=np.uint32) + ks[1]
    for i in range(5):
        for r in rot[i % 2]:
            x0 = (x0 + x1).astype(np.uint32)
            x1 = rotl(x1, r)
            x1 = (x0 ^ x1).astype(np.uint32)
        x0 = (x0 + ks[(i + 1) % 3]).astype(np.uint32)
        x1 = (x1 + ks[(i + 2) % 3] + np.uint32(i + 1)).astype(np.uint32)
    bits = (x0 ^ x1).astype(np.uint32)
    f = ((bits >> np.uint32(9)) | np.uint32(0x3F800000)).view(np.float32)
    f = (f - np.float32(1.0)).astype(np.float32)
    tiny = np.float32(np.finfo(np.float32).tiny)
    u = np.maximum(tiny, (f * np.float32(1.0) + tiny).astype(np.float32))
    return (-np.log(-np.log(u.astype(np.float64)))).astype(np.float32)


_G_NP = _baked_gumbel()

_mesh = plsc.VectorSubcoreMesh(core_axis_name="c", subcore_axis_name="s")


def _worker_id():
    return lax.axis_index("s") * _NC + lax.axis_index("c")


def _scan_body(z_hbm, w_hbm, g_hbm, vals_hbm, idxs_hbm,
               z_v, w_buf0, w_buf1, g_buf0, g_buf1,
               stage_v, stage_i, sem0, sem1):
    wid = _worker_id()
    pltpu.sync_copy(z_hbm, z_v)
    neg = jnp.full((_L,), -jnp.inf, jnp.float32)
    zero = jnp.zeros((_L,), jnp.int32)
    iota = lax.iota(jnp.int32, _L)
    zvec = z_v[...]
    zk = [jnp.sum(jnp.where(iota == k, zvec, 0.0)) for k in range(_D)]

    w_bufs = (w_buf0, w_buf1)
    g_bufs = (g_buf0, g_buf1)
    sems = (sem0, sem1)

    def start_dma(t):
        c0 = (wid + t * _NW) * _CHUNK
        b = t % 2
        pltpu.make_async_copy(w_hbm.at[:, pl.ds(c0, _CHUNK)], w_bufs[b],
                              sems[b]).start()
        pltpu.make_async_copy(g_hbm.at[pl.ds(c0, _CHUNK)], g_bufs[b],
                              sems[b]).start()

    start_dma(0)
    bv, bi = neg, zero
    for t in range(_TRIPS):
        if t + 1 < _TRIPS:
            start_dma(t + 1)
        b = t % 2
        c0 = (wid + t * _NW) * _CHUNK
        pltpu.make_async_copy(w_hbm.at[:, pl.ds(c0, _CHUNK)], w_bufs[b],
                              sems[b]).wait()
        pltpu.make_async_copy(g_hbm.at[pl.ds(c0, _CHUNK)], g_bufs[b],
                              sems[b]).wait()
        w_buf = w_bufs[b]
        g_buf = g_bufs[b]

        @pl.loop(0, _VPC, init_carry=(bv, bi), unroll=4)
        def inner(i, car):
            v, ix = car
            base = i * _L
            acc = g_buf[pl.ds(base, _L)]
            for k in range(_D):
                acc = acc + zk[k] * w_buf[k, pl.ds(base, _L)]
            idxv = (c0 + base) + iota
            m = acc > v
            return jnp.where(m, acc, v), jnp.where(m, idxv, ix)

        bv, bi = inner

    stage_v[...] = bv
    stage_i[...] = bi
    pltpu.sync_copy(stage_v, vals_hbm.at[pl.ds(wid * _L, _L)])
    pltpu.sync_copy(stage_i, idxs_hbm.at[pl.ds(wid * _L, _L)])


def _tc_body(z_ref, w_ref, g_ref, val_ref, idx_ref, bv_ref, bi_ref):
    i = pl.program_id(0)
    zb = jnp.broadcast_to(z_ref[...].reshape(1, _D), (8, _D))
    scores = lax.dot_general(zb, w_ref[...], (((1,), (0,)), ((), ())),
                             preferred_element_type=jnp.float32)
    g = jnp.broadcast_to(g_ref[...].reshape(1, _TBLK), (8, _TBLK))
    pos = (_S + i * _TBLK) + lax.broadcasted_iota(jnp.int32, (8, _TBLK), 1)
    sc = jnp.where(pos < _N, scores + g, -jnp.inf)

    @pl.when(i == 0)
    def _():
        bv_ref[...] = sc
        bi_ref[...] = pos

    @pl.when(i > 0)
    def _():
        m = sc > bv_ref[...]
        bv_ref[...] = jnp.where(m, sc, bv_ref[...])
        bi_ref[...] = jnp.where(m, pos, bi_ref[...])

    @pl.when(i == _TGRID - 1)
    def _():
        v = bv_ref[...]
        mmax = jnp.max(v)
        win = jnp.max(jnp.where(v == mmax, bi_ref[...], -1))
        val_ref[...] = jnp.full((_L,), mmax, jnp.float32)
        idx_ref[...] = jnp.full((_L,), win, jnp.int32)


def _pick_body(vals_hbm, idxs_hbm, tcv_hbm, tci_hbm, xt_hbm, out_hbm,
               v_buf, i_buf, tv_buf, ti_buf, win_v, row_v, sem):
    wid = _worker_id()

    @pl.when(wid == 0)
    def _():
        pltpu.sync_copy(vals_hbm, v_buf)
        pltpu.sync_copy(idxs_hbm, i_buf)
        pltpu.sync_copy(tcv_hbm, tv_buf)
        pltpu.sync_copy(tci_hbm, ti_buf)
        mv = v_buf[pl.ds(0, _L)]
        mi = i_buf[pl.ds(0, _L)]
        for w in range(1, _NW):
            av = v_buf[pl.ds(w * _L, _L)]
            ai = i_buf[pl.ds(w * _L, _L)]
            m = av > mv
            mv = jnp.where(m, av, mv)
            mi = jnp.where(m, ai, mi)
        av = tv_buf[...]
        ai = ti_buf[...]
        m = av > mv
        mv = jnp.where(m, av, mv)
        mi = jnp.where(m, ai, mi)
        mmax = jnp.max(mv)
        win = jnp.max(jnp.where(mv == mmax, mi, -1))
        blk = pl.multiple_of(win & ~jnp.int32(127), 128)
        gather = pltpu.make_async_copy(xt_hbm.at[:, pl.ds(blk, 128)], row_v, sem)
        gather.start()
        gather.wait()
        sub = jnp.full((_L,), win & jnp.int32(127), jnp.int32)
        col = plsc.load_gather(row_v, [lax.iota(jnp.int32, _L), sub])
        win_v[...] = col
        pltpu.sync_copy(win_v, out_hbm)


_scan = pl.kernel(
    _scan_body,
    out_type=(jax.ShapeDtypeStruct((_NW * _L,), jnp.float32),
              jax.ShapeDtypeStruct((_NW * _L,), jnp.int32)),
    mesh=_mesh,
    compiler_params=pltpu.CompilerParams(needs_layout_passes=False),
    scratch_types=[
        pltpu.VMEM((_D,), jnp.float32),
        pltpu.VMEM((_D, _CHUNK), jnp.float32),
        pltpu.VMEM((_D, _CHUNK), jnp.float32),
        pltpu.VMEM((_CHUNK,), jnp.float32),
        pltpu.VMEM((_CHUNK,), jnp.float32),
        pltpu.VMEM((_L,), jnp.float32),
        pltpu.VMEM((_L,), jnp.int32),
        pltpu.SemaphoreType.DMA,
        pltpu.SemaphoreType.DMA,
    ],
)

_tcscan = pl.pallas_call(
    _tc_body,
    grid=(_TGRID,),
    in_specs=[
        pl.BlockSpec((_D,), lambda i: (0,)),
        pl.BlockSpec((_D, _TBLK), lambda i: (0, _S // _TBLK + i)),
        pl.BlockSpec((_TBLK,), lambda i: (_S // _TBLK + i,)),
    ],
    out_specs=[
        pl.BlockSpec((_L,), lambda i: (0,)),
        pl.BlockSpec((_L,), lambda i: (0,)),
    ],
    out_shape=(jax.ShapeDtypeStruct((_L,), jnp.float32),
               jax.ShapeDtypeStruct((_L,), jnp.int32)),
    scratch_shapes=[
        pltpu.VMEM((8, _TBLK), jnp.float32),
        pltpu.VMEM((8, _TBLK), jnp.int32),
    ],
)


def kernel(z, x, W):
    g = jnp.asarray(_G_NP)
    vals, idxs = _scan(z, W, g)
    tcv, tci = _tcscan(z, W, g)
    row = _pick(vals, idxs, tcv, tci, x.T)
    return row.reshape(1, _D)


_pick = pl.kernel(
    _pick_body,
    out_type=jax.ShapeDtypeStruct((_D,), jnp.float32),
    mesh=_mesh,
    compiler_params=pltpu.CompilerParams(needs_layout_passes=False),
    scratch_types=[
        pltpu.VMEM((_NW * _L,), jnp.float32),
        pltpu.VMEM((_NW * _L,), jnp.int32),
        pltpu.VMEM((_L,), jnp.float32),
        pltpu.VMEM((_L,), jnp.int32),
        pltpu.VMEM((_L,), jnp.float32),
        pltpu.VMEM((_D, 128), jnp.float32),
        pltpu.SemaphoreType.DMA,
    ],
)


# speedup vs baseline: 3.8415x; 3.1025x over previous
import numpy as np
import jax
import jax.numpy as jnp
from jax import lax
from jax.experimental import pallas as pl
from jax.experimental.pallas import tpu as pltpu
from jax.experimental.pallas import tpu_sc as plsc

_N = 1_000_000
_D = 16
_NC = 2
_NS = 16
_L = 16
_NW = _NC * _NS
_CHUNK = 2048
_S = 192 * 2048
_NCHUNKS = _S // _CHUNK
_TRIPS = (_NCHUNKS + _NW - 1) // _NW
_VPC = _CHUNK // _L
_TBLK = 16384
_TGRID = (_N - _S + _TBLK - 1) // _TBLK


def _baked_gumbel() -> np.ndarray:
    def rotl(x, r):
        return ((x << np.uint32(r)) | (x >> np.uint32(32 - r))).astype(np.uint32)

    k0 = np.uint32(0)
    k1 = np.uint32(42)
    ks = [k0, k1, np.uint32(np.uint32(k0 ^ k1) ^ np.uint32(0x1BD11BDA))]
    rot = [(13, 15, 26, 6), (17, 29, 16, 24)]
    x0 = np.zeros(_N, np.uint32) + ks[0]
    x1 = np.arange(_N, dtype=np.uint32) + ks[1]
    for i in range(5):
        for r in rot[i % 2]:
            x0 = (x0 + x1).astype(np.uint32)
            x1 = rotl(x1, r)
            x1 = (x0 ^ x1).astype(np.uint32)
        x0 = (x0 + ks[(i + 1) % 3]).astype(np.uint32)
        x1 = (x1 + ks[(i + 2) % 3] + np.uint32(i + 1)).astype(np.uint32)
    bits = (x0 ^ x1).astype(np.uint32)
    f = ((bits >> np.uint32(9)) | np.uint32(0x3F800000)).view(np.float32)
    f = (f - np.float32(1.0)).astype(np.float32)
    tiny = np.float32(np.finfo(np.float32).tiny)
    u = np.maximum(tiny, (f * np.float32(1.0) + tiny).astype(np.float32))
    return (-np.log(-np.log(u.astype(np.float64)))).astype(np.float32)


_G_NP = _baked_gumbel()

_mesh = plsc.VectorSubcoreMesh(core_axis_name="c", subcore_axis_name="s")


def _worker_id():
    return lax.axis_index("s") * _NC + lax.axis_index("c")


def _scan_body(z_hbm, w_hbm, g_hbm, vals_hbm, idxs_hbm,
               z_v, w_buf0, w_buf1, g_buf0, g_buf1,
               stage_v, stage_i, sem0, sem1):
    wid = _worker_id()
    pltpu.sync_copy(z_hbm, z_v)
    neg = jnp.full((_L,), -jnp.inf, jnp.float32)
    zero = jnp.zeros((_L,), jnp.int32)
    iota = lax.iota(jnp.int32, _L)
    zvec = z_v[...]
    zk = [jnp.sum(jnp.where(iota == k, zvec, 0.0)) for k in range(_D)]

    w_bufs = (w_buf0, w_buf1)
    g_bufs = (g_buf0, g_buf1)
    sems = (sem0, sem1)

    def start_dma(t):
        c0 = (wid + t * _NW) * _CHUNK
        b = t % 2
        pltpu.make_async_copy(w_hbm.at[:, pl.ds(c0, _CHUNK)], w_bufs[b],
                              sems[b]).start()
        pltpu.make_async_copy(g_hbm.at[pl.ds(c0, _CHUNK)], g_bufs[b],
                              sems[b]).start()

    start_dma(0)
    bv, bi = neg, zero
    for t in range(_TRIPS):
        if t + 1 < _TRIPS:
            start_dma(t + 1)
        b = t % 2
        c0 = (wid + t * _NW) * _CHUNK
        pltpu.make_async_copy(w_hbm.at[:, pl.ds(c0, _CHUNK)], w_bufs[b],
                              sems[b]).wait()
        pltpu.make_async_copy(g_hbm.at[pl.ds(c0, _CHUNK)], g_bufs[b],
                              sems[b]).wait()
        w_buf = w_bufs[b]
        g_buf = g_bufs[b]

        @pl.loop(0, _VPC, init_carry=(bv, bi), unroll=4)
        def inner(i, car):
            v, ix = car
            base = i * _L
            acc = g_buf[pl.ds(base, _L)]
            for k in range(_D):
                acc = acc + zk[k] * w_buf[k, pl.ds(base, _L)]
            idxv = (c0 + base) + iota
            m = acc > v
            return jnp.where(m, acc, v), jnp.where(m, idxv, ix)

        bv, bi = inner

    stage_v[...] = bv
    stage_i[...] = bi
    pltpu.sync_copy(stage_v, vals_hbm.at[pl.ds(wid * _L, _L)])
    pltpu.sync_copy(stage_i, idxs_hbm.at[pl.ds(wid * _L, _L)])


def _tc_body(z_ref, w_ref, g_ref, val_ref, idx_ref, bv_ref, bi_ref):
    i = pl.program_id(0)
    zb = jnp.broadcast_to(z_ref[...].reshape(1, _D), (8, _D))
    scores = lax.dot_general(zb, w_ref[...], (((1,), (0,)), ((), ())),
                             preferred_element_type=jnp.float32)
    g = jnp.broadcast_to(g_ref[...].reshape(1, _TBLK), (8, _TBLK))
    pos = (_S + i * _TBLK) + lax.broadcasted_iota(jnp.int32, (8, _TBLK), 1)
    sc = jnp.where(pos < _N, scores + g, -jnp.inf)

    @pl.when(i == 0)
    def _():
        bv_ref[...] = sc
        bi_ref[...] = pos

    @pl.when(i > 0)
    def _():
        m = sc > bv_ref[...]
        bv_ref[...] = jnp.where(m, sc, bv_ref[...])
        bi_ref[...] = jnp.where(m, pos, bi_ref[...])

    @pl.when(i == _TGRID - 1)
    def _():
        v = bv_ref[...]
        mmax = jnp.max(v)
        win = jnp.max(jnp.where(v == mmax, bi_ref[...], -1))
        val_ref[...] = jnp.full((_L,), mmax, jnp.float32)
        idx_ref[...] = jnp.full((_L,), win, jnp.int32)


def _pick_body(vals_hbm, idxs_hbm, tcv_hbm, tci_hbm, xt_hbm, out_hbm,
               v_buf, i_buf, tv_buf, ti_buf, win_v, row_v, sem):
    wid = _worker_id()

    @pl.when(wid == 0)
    def _():
        pltpu.sync_copy(vals_hbm, v_buf)
        pltpu.sync_copy(idxs_hbm, i_buf)
        pltpu.sync_copy(tcv_hbm, tv_buf)
        pltpu.sync_copy(tci_hbm, ti_buf)
        mv = v_buf[pl.ds(0, _L)]
        mi = i_buf[pl.ds(0, _L)]
        for w in range(1, _NW):
            av = v_buf[pl.ds(w * _L, _L)]
            ai = i_buf[pl.ds(w * _L, _L)]
            m = av > mv
            mv = jnp.where(m, av, mv)
            mi = jnp.where(m, ai, mi)
        av = tv_buf[...]
        ai = ti_buf[...]
        m = av > mv
        mv = jnp.where(m, av, mv)
        mi = jnp.where(m, ai, mi)
        mmax = jnp.max(mv)
        win = jnp.max(jnp.where(mv == mmax, mi, -1))
        blk = pl.multiple_of(win & ~jnp.int32(127), 128)
        gather = pltpu.make_async_copy(xt_hbm.at[:, pl.ds(blk, 128)], row_v, sem)
        gather.start()
        gather.wait()
        sub = jnp.full((_L,), win & jnp.int32(127), jnp.int32)
        col = plsc.load_gather(row_v, [lax.iota(jnp.int32, _L), sub])
        win_v[...] = col
        pltpu.sync_copy(win_v, out_hbm)


_scan = pl.kernel(
    _scan_body,
    out_type=(jax.ShapeDtypeStruct((_NW * _L,), jnp.float32),
              jax.ShapeDtypeStruct((_NW * _L,), jnp.int32)),
    mesh=_mesh,
    compiler_params=pltpu.CompilerParams(needs_layout_passes=False),
    scratch_types=[
        pltpu.VMEM((_D,), jnp.float32),
        pltpu.VMEM((_D, _CHUNK), jnp.float32),
        pltpu.VMEM((_D, _CHUNK), jnp.float32),
        pltpu.VMEM((_CHUNK,), jnp.float32),
        pltpu.VMEM((_CHUNK,), jnp.float32),
        pltpu.VMEM((_L,), jnp.float32),
        pltpu.VMEM((_L,), jnp.int32),
        pltpu.SemaphoreType.DMA,
        pltpu.SemaphoreType.DMA,
    ],
)

_tcscan = pl.pallas_call(
    _tc_body,
    grid=(_TGRID,),
    in_specs=[
        pl.BlockSpec((_D,), lambda i: (0,)),
        pl.BlockSpec((_D, _TBLK), lambda i: (0, _S // _TBLK + i)),
        pl.BlockSpec((_TBLK,), lambda i: (_S // _TBLK + i,)),
    ],
    out_specs=[
        pl.BlockSpec((_L,), lambda i: (0,)),
        pl.BlockSpec((_L,), lambda i: (0,)),
    ],
    out_shape=(jax.ShapeDtypeStruct((_L,), jnp.float32),
               jax.ShapeDtypeStruct((_L,), jnp.int32)),
    scratch_shapes=[
        pltpu.VMEM((8, _TBLK), jnp.float32),
        pltpu.VMEM((8, _TBLK), jnp.int32),
    ],
)


def kernel(z, x, W):
    g = jnp.asarray(_G_NP)
    vals, idxs = _scan(z, W, g)
    tcv, tci = _tcscan(z, W, g)
    row = _pick(vals, idxs, tcv, tci, x.T)
    return row.reshape(1, _D)


_pick = pl.kernel(
    _pick_body,
    out_type=jax.ShapeDtypeStruct((_D,), jnp.float32),
    mesh=_mesh,
    compiler_params=pltpu.CompilerParams(needs_layout_passes=False),
    scratch_types=[
        pltpu.VMEM((_NW * _L,), jnp.float32),
        pltpu.VMEM((_NW * _L,), jnp.int32),
        pltpu.VMEM((_L,), jnp.float32),
        pltpu.VMEM((_L,), jnp.int32),
        pltpu.VMEM((_L,), jnp.float32),
        pltpu.VMEM((_D, 128), jnp.float32),
        pltpu.SemaphoreType.DMA,
    ],
)


# speedup vs baseline: 4.4243x; 1.1517x over previous
import numpy as np
import jax
import jax.numpy as jnp
from jax import lax
from jax.experimental import pallas as pl
from jax.experimental.pallas import tpu as pltpu
from jax.experimental.pallas import tpu_sc as plsc

_N = 1_000_000
_D = 16
_NC = 2
_NS = 16
_L = 16
_NW = _NC * _NS
_CHUNK = 2048
_S = 288 * 2048
_NCHUNKS = _S // _CHUNK
_TRIPS = (_NCHUNKS + _NW - 1) // _NW
_VPC = _CHUNK // _L
_TBLK = 16384
_TGRID = (_N - _S + _TBLK - 1) // _TBLK


def _baked_gumbel() -> np.ndarray:
    def rotl(x, r):
        return ((x << np.uint32(r)) | (x >> np.uint32(32 - r))).astype(np.uint32)

    k0 = np.uint32(0)
    k1 = np.uint32(42)
    ks = [k0, k1, np.uint32(np.uint32(k0 ^ k1) ^ np.uint32(0x1BD11BDA))]
    rot = [(13, 15, 26, 6), (17, 29, 16, 24)]
    x0 = np.zeros(_N, np.uint32) + ks[0]
    x1 = np.arange(_N, dtype=np.uint32) + ks[1]
    for i in range(5):
        for r in rot[i % 2]:
            x0 = (x0 + x1).astype(np.uint32)
            x1 = rotl(x1, r)
            x1 = (x0 ^ x1).astype(np.uint32)
        x0 = (x0 + ks[(i + 1) % 3]).astype(np.uint32)
        x1 = (x1 + ks[(i + 2) % 3] + np.uint32(i + 1)).astype(np.uint32)
    bits = (x0 ^ x1).astype(np.uint32)
    f = ((bits >> np.uint32(9)) | np.uint32(0x3F800000)).view(np.float32)
    f = (f - np.float32(1.0)).astype(np.float32)
    tiny = np.float32(np.finfo(np.float32).tiny)
    u = np.maximum(tiny, (f * np.float32(1.0) + tiny).astype(np.float32))
    return (-np.log(-np.log(u.astype(np.float64)))).astype(np.float32)


_G_NP = _baked_gumbel()

_mesh = plsc.VectorSubcoreMesh(core_axis_name="c", subcore_axis_name="s")


def _worker_id():
    return lax.axis_index("s") * _NC + lax.axis_index("c")


def _scan_body(z_hbm, w_hbm, g_hbm, vals_hbm, idxs_hbm,
               z_v, w_buf0, w_buf1, g_buf0, g_buf1,
               stage_v, stage_i, sem0, sem1):
    wid = _worker_id()
    pltpu.sync_copy(z_hbm, z_v)
    neg = jnp.full((_L,), -jnp.inf, jnp.float32)
    zero = jnp.zeros((_L,), jnp.int32)
    iota = lax.iota(jnp.int32, _L)
    zvec = z_v[...]
    zk = [jnp.sum(jnp.where(iota == k, zvec, 0.0)) for k in range(_D)]

    w_bufs = (w_buf0, w_buf1)
    g_bufs = (g_buf0, g_buf1)
    sems = (sem0, sem1)

    def start_dma(t):
        c0 = (wid + t * _NW) * _CHUNK
        b = t % 2
        pltpu.make_async_copy(w_hbm.at[:, pl.ds(c0, _CHUNK)], w_bufs[b],
                              sems[b]).start()
        pltpu.make_async_copy(g_hbm.at[pl.ds(c0, _CHUNK)], g_bufs[b],
                              sems[b]).start()

    start_dma(0)
    bv, bi = neg, zero
    for t in range(_TRIPS):
        if t + 1 < _TRIPS:
            start_dma(t + 1)
        b = t % 2
        c0 = (wid + t * _NW) * _CHUNK
        pltpu.make_async_copy(w_hbm.at[:, pl.ds(c0, _CHUNK)], w_bufs[b],
                              sems[b]).wait()
        pltpu.make_async_copy(g_hbm.at[pl.ds(c0, _CHUNK)], g_bufs[b],
                              sems[b]).wait()
        w_buf = w_bufs[b]
        g_buf = g_bufs[b]

        @pl.loop(0, _VPC, init_carry=(bv, bi), unroll=4)
        def inner(i, car):
            v, ix = car
            base = i * _L
            acc = g_buf[pl.ds(base, _L)]
            for k in range(_D):
                acc = acc + zk[k] * w_buf[k, pl.ds(base, _L)]
            idxv = (c0 + base) + iota
            m = acc > v
            return jnp.where(m, acc, v), jnp.where(m, idxv, ix)

        bv, bi = inner

    stage_v[...] = bv
    stage_i[...] = bi
    pltpu.sync_copy(stage_v, vals_hbm.at[pl.ds(wid * _L, _L)])
    pltpu.sync_copy(stage_i, idxs_hbm.at[pl.ds(wid * _L, _L)])


def _tc_body(z_ref, w_ref, g_ref, val_ref, idx_ref, bv_ref, bi_ref):
    i = pl.program_id(0)
    zb = jnp.broadcast_to(z_ref[...].reshape(1, _D), (8, _D))
    scores = lax.dot_general(zb, w_ref[...], (((1,), (0,)), ((), ())),
                             preferred_element_type=jnp.float32)
    g = jnp.broadcast_to(g_ref[...].reshape(1, _TBLK), (8, _TBLK))
    pos = (_S + i * _TBLK) + lax.broadcasted_iota(jnp.int32, (8, _TBLK), 1)
    sc = jnp.where(pos < _N, scores + g, -jnp.inf)

    @pl.when(i == 0)
    def _():
        bv_ref[...] = sc
        bi_ref[...] = pos

    @pl.when(i > 0)
    def _():
        m = sc > bv_ref[...]
        bv_ref[...] = jnp.where(m, sc, bv_ref[...])
        bi_ref[...] = jnp.where(m, pos, bi_ref[...])

    @pl.when(i == _TGRID - 1)
    def _():
        v = bv_ref[...]
        mmax = jnp.max(v)
        win = jnp.max(jnp.where(v == mmax, bi_ref[...], -1))
        val_ref[...] = jnp.full((_L,), mmax, jnp.float32)
        idx_ref[...] = jnp.full((_L,), win, jnp.int32)


def _pick_body(vals_hbm, idxs_hbm, tcv_hbm, tci_hbm, xt_hbm, out_hbm,
               v_buf, i_buf, tv_buf, ti_buf, win_v, row_v, sem):
    wid = _worker_id()

    @pl.when(wid == 0)
    def _():
        pltpu.sync_copy(vals_hbm, v_buf)
        pltpu.sync_copy(idxs_hbm, i_buf)
        pltpu.sync_copy(tcv_hbm, tv_buf)
        pltpu.sync_copy(tci_hbm, ti_buf)
        mv = v_buf[pl.ds(0, _L)]
        mi = i_buf[pl.ds(0, _L)]
        for w in range(1, _NW):
            av = v_buf[pl.ds(w * _L, _L)]
            ai = i_buf[pl.ds(w * _L, _L)]
            m = av > mv
            mv = jnp.where(m, av, mv)
            mi = jnp.where(m, ai, mi)
        av = tv_buf[...]
        ai = ti_buf[...]
        m = av > mv
        mv = jnp.where(m, av, mv)
        mi = jnp.where(m, ai, mi)
        mmax = jnp.max(mv)
        win = jnp.max(jnp.where(mv == mmax, mi, -1))
        blk = pl.multiple_of(win & ~jnp.int32(127), 128)
        gather = pltpu.make_async_copy(xt_hbm.at[:, pl.ds(blk, 128)], row_v, sem)
        gather.start()
        gather.wait()
        sub = jnp.full((_L,), win & jnp.int32(127), jnp.int32)
        col = plsc.load_gather(row_v, [lax.iota(jnp.int32, _L), sub])
        win_v[...] = col
        pltpu.sync_copy(win_v, out_hbm)


_scan = pl.kernel(
    _scan_body,
    out_type=(jax.ShapeDtypeStruct((_NW * _L,), jnp.float32),
              jax.ShapeDtypeStruct((_NW * _L,), jnp.int32)),
    mesh=_mesh,
    compiler_params=pltpu.CompilerParams(needs_layout_passes=False),
    scratch_types=[
        pltpu.VMEM((_D,), jnp.float32),
        pltpu.VMEM((_D, _CHUNK), jnp.float32),
        pltpu.VMEM((_D, _CHUNK), jnp.float32),
        pltpu.VMEM((_CHUNK,), jnp.float32),
        pltpu.VMEM((_CHUNK,), jnp.float32),
        pltpu.VMEM((_L,), jnp.float32),
        pltpu.VMEM((_L,), jnp.int32),
        pltpu.SemaphoreType.DMA,
        pltpu.SemaphoreType.DMA,
    ],
)

_tcscan = pl.pallas_call(
    _tc_body,
    grid=(_TGRID,),
    in_specs=[
        pl.BlockSpec((_D,), lambda i: (0,)),
        pl.BlockSpec((_D, _TBLK), lambda i: (0, _S // _TBLK + i)),
        pl.BlockSpec((_TBLK,), lambda i: (_S // _TBLK + i,)),
    ],
    out_specs=[
        pl.BlockSpec((_L,), lambda i: (0,)),
        pl.BlockSpec((_L,), lambda i: (0,)),
    ],
    out_shape=(jax.ShapeDtypeStruct((_L,), jnp.float32),
               jax.ShapeDtypeStruct((_L,), jnp.int32)),
    scratch_shapes=[
        pltpu.VMEM((8, _TBLK), jnp.float32),
        pltpu.VMEM((8, _TBLK), jnp.int32),
    ],
)


def kernel(z, x, W):
    g = jnp.asarray(_G_NP)
    vals, idxs = _scan(z, W, g)
    tcv, tci = _tcscan(z, W, g)
    row = _pick(vals, idxs, tcv, tci, x.T)
    return row.reshape(1, _D)


_pick = pl.kernel(
    _pick_body,
    out_type=jax.ShapeDtypeStruct((_D,), jnp.float32),
    mesh=_mesh,
    compiler_params=pltpu.CompilerParams(needs_layout_passes=False),
    scratch_types=[
        pltpu.VMEM((_NW * _L,), jnp.float32),
        pltpu.VMEM((_NW * _L,), jnp.int32),
        pltpu.VMEM((_L,), jnp.float32),
        pltpu.VMEM((_L,), jnp.int32),
        pltpu.VMEM((_L,), jnp.float32),
        pltpu.VMEM((_D, 128), jnp.float32),
        pltpu.SemaphoreType.DMA,
    ],
)
